# baseline (speedup 1.0000x reference)
import jax
import jax.numpy as jnp
from jax import lax
from jax.experimental import pallas as pl
from jax.experimental.pallas import tpu as pltpu
from jax.experimental.pallas import tpu_sc as plsc

N, D, C = 10000, 128, 10
E = 320000
NW = 32
CHUNK = 128
CPW = 80
EP = NW * CPW * CHUNK
NPAD = 10240
BR = 1024
GRID = NPAD // BR
K1 = 8000
K2 = 6400


def _sc_agg_body(x_hbm, src_hbm, dst_hbm, z_hbm, out_hbm,
                 acc, src_v, dst_v, rows0, rows1, sem0, sem1):
    cid = lax.axis_index("c")
    sid = lax.axis_index("s")
    wid = sid * 2 + cid
    zr = NPAD // 16
    pltpu.sync_copy(z_hbm.at[pl.ds(sid * zr, zr)], acc.at[pl.ds(sid * zr, zr)])
    plsc.subcore_barrier()
    HC = CPW // 2
    for h in range(2):
        base = wid * CPW + h * HC
        pltpu.sync_copy(src_hbm.at[pl.ds(base, HC)], src_v)
        pltpu.sync_copy(dst_hbm.at[pl.ds(base, HC)], dst_v)
        pltpu.async_copy(x_hbm.at[src_v.at[0]], rows0, sem0)

        def pair(i, carry):
            j0 = 2 * i
            j1 = j0 + 1
            pltpu.async_copy(x_hbm.at[src_v.at[j1]], rows1, sem1)
            pltpu.make_async_copy(x_hbm.at[src_v.at[j0]], rows0, sem0).wait()
            pltpu.sync_copy(rows0, acc.at[dst_v.at[j0]], add=True)
            jn = jnp.where(j0 + 2 < HC, j0 + 2, 0)
            pltpu.async_copy(x_hbm.at[src_v.at[jn]], rows0, sem0)
            pltpu.make_async_copy(x_hbm.at[src_v.at[j1]], rows1, sem1).wait()
            pltpu.sync_copy(rows1, acc.at[dst_v.at[j1]], add=True)
            return carry

        lax.fori_loop(0, HC // 2, pair, 0)
        pltpu.make_async_copy(x_hbm.at[src_v.at[0]], rows0, sem0).wait()
    plsc.subcore_barrier()
    pltpu.sync_copy(acc.at[pl.ds(sid * zr, zr)],
                    out_hbm.at[pl.ds(cid * NPAD + sid * zr, zr)])


_sc_agg_fn = None


def _sc_agg(x, srcp, dstp, z):
    global _sc_agg_fn
    if _sc_agg_fn is None:
        _sc_agg_fn = pl.kernel(
            _sc_agg_body,
            out_type=jax.ShapeDtypeStruct((2 * NPAD, D), jnp.float32),
            mesh=plsc.VectorSubcoreMesh(core_axis_name="c", subcore_axis_name="s"),
            scratch_types=[
                pltpu.VMEM_SHARED((NPAD, D), jnp.float32),
                pltpu.VMEM((CPW // 2, CHUNK), jnp.int32),
                pltpu.VMEM((CPW // 2, CHUNK), jnp.int32),
                pltpu.VMEM((CHUNK, D), jnp.float32),
                pltpu.VMEM((CHUNK, D), jnp.float32),
                pltpu.SemaphoreType.DMA,
                pltpu.SemaphoreType.DMA,
            ],
        )
    return _sc_agg_fn(x, srcp, dstp, z)


def _u32_key(s):
    i = lax.bitcast_convert_type(s, jnp.int32)
    u = lax.bitcast_convert_type(s, jnp.uint32)
    return jnp.where(i < 0, ~u, u | jnp.uint32(0x80000000))


def _make_conv_body(mask, resid, score, final):
    def body(*refs):
        it = iter(refs)
        a0 = next(it)[...]
        a1 = next(it)[...]
        x = next(it)[...]
        wn = next(it)[...]
        wr = next(it)[...]
        b = next(it)[...]
        m = next(it)[...] if mask else None
        r = next(it)[...] if resid else None
        pm = next(it)[...] if score else None
        pn = next(it)[...] if score else None
        wl = next(it)[...] if final else None
        blv = next(it)[...] if final else None
        out_ref = next(it)
        s_ref = next(it) if score else None
        z = jnp.dot(a0 + a1, wn, preferred_element_type=jnp.float32)
        z = z + jnp.dot(x, wr, preferred_element_type=jnp.float32)
        z = jnp.maximum(z + b[0:1, :], 0.0)
        if mask:
            z = z * m
        if resid:
            z = z + r
        if final:
            lg = jnp.dot(z, wl, preferred_element_type=jnp.float32) + blv[0:1, :]
            lg = lg - jnp.max(lg, axis=-1, keepdims=True)
            z = lg - jnp.log(jnp.sum(jnp.exp(lg), axis=-1, keepdims=True))
        out_ref[...] = z
        if score:
            s_ref[...] = jnp.tanh(
                jnp.dot(z, pm, preferred_element_type=jnp.float32) / pn[0:1, :])
    return body


_ROW = pl.BlockSpec((BR, D), lambda i: (i, 0))
_ROW1 = pl.BlockSpec((BR, D), lambda i: (i + GRID, 0))
_W = pl.BlockSpec((D, D), lambda i: (0, 0))
_B8 = pl.BlockSpec((8, D), lambda i: (0, 0))


def _conv(P, x, wn, wr, b, m=None, r=None, pv=None, wl=None, blv=None):
    score, final = pv is not None, wl is not None
    ins = [P, P, x, wn, wr, jnp.broadcast_to(b[None, :], (8, D))]
    specs = [_ROW, _ROW1, _ROW, _W, _W, _B8]
    if m is not None:
        ins.append(m); specs.append(_ROW)
    if r is not None:
        ins.append(r); specs.append(_ROW)
    if score:
        ins.append(jnp.broadcast_to(pv[:, None], (D, D)))
        ins.append(jnp.broadcast_to(jnp.linalg.norm(pv)[None, None], (8, D)))
        specs.extend([_W, _B8])
    if final:
        ins.extend([wl, jnp.broadcast_to(blv[None, :], (8, D))])
        specs.extend([_W, _B8])
    out_shape = jax.ShapeDtypeStruct((NPAD, D), jnp.float32)
    out_spec = _ROW
    if score:
        out_shape = [out_shape, jax.ShapeDtypeStruct((NPAD, D), jnp.float32)]
        out_spec = [_ROW, _ROW]
    return pl.pallas_call(
        _make_conv_body(m is not None, r is not None, score, final),
        grid=(GRID,),
        in_specs=specs,
        out_specs=out_spec,
        out_shape=out_shape,
    )(*ins)


def _select_body(s_ref, k_ref, t_ref):
    keys = _u32_key(s_ref[...])
    kk = k_ref[0, 0]

    def step(t, prefix):
        cand = prefix | (jnp.uint32(1) << (jnp.uint32(31) - t.astype(jnp.uint32)))
        cnt = jnp.sum((keys >= cand).astype(jnp.int32))
        return jnp.where(cnt >= kk, cand, prefix)

    tk = lax.fori_loop(0, 32, step, jnp.uint32(0))
    t_ref[0, 0] = lax.bitcast_convert_type(tk, jnp.int32)


def _select(s, kk):
    kk_arr = jnp.asarray(kk, jnp.int32).reshape(1, 1)
    return pl.pallas_call(
        _select_body,
        in_specs=[pl.BlockSpec(memory_space=pltpu.VMEM),
                  pl.BlockSpec(memory_space=pltpu.SMEM)],
        out_specs=pl.BlockSpec(memory_space=pltpu.SMEM),
        out_shape=jax.ShapeDtypeStruct((1, 1), jnp.int32),
    )(s.reshape(NPAD // 128, 128), kk_arr)[0, 0]


def _scale_body(x_ref, rs_ref, o_ref):
    o_ref[...] = x_ref[...] * rs_ref[...]


def _scale(x, rs):
    return pl.pallas_call(
        _scale_body,
        grid=(GRID,),
        in_specs=[_ROW, _ROW],
        out_specs=_ROW,
        out_shape=jax.ShapeDtypeStruct((NPAD, D), jnp.float32),
    )(x, rs)


def _pool(xout, sout, kk, s_prev=None):
    s = jnp.where(lax.iota(jnp.int32, NPAD) < N, sout[:, 0], -jnp.inf)
    keys = _u32_key(s)
    tu = lax.bitcast_convert_type(_select(s, kk), jnp.uint32)
    gt = keys > tu
    eq = keys == tu
    need = kk - jnp.sum(gt.astype(jnp.int32))
    if s_prev is None:
        kept = gt | (eq & (jnp.cumsum(eq.astype(jnp.int32)) <= need))
    else:
        sp = jnp.where(eq, s_prev, -jnp.inf)
        kp = _u32_key(sp)
        t1 = lax.bitcast_convert_type(_select(sp, need), jnp.uint32)
        gt2 = kp > t1
        eq2 = kp == t1
        need2 = need - jnp.sum(gt2.astype(jnp.int32))
        kept = gt | gt2 | (eq2 & (jnp.cumsum(eq2.astype(jnp.int32)) <= need2))
    rs = jnp.where(kept[:, None], sout, 0.0)
    xn = _scale(xout, rs)
    km = jnp.broadcast_to(kept[:, None].astype(jnp.float32), (NPAD, D))
    return kept, xn, km


def kernel(x, g, Wr1, Wn1, b1, p1, Wr2, Wn2, b2, p2, Wr3, Wn3, b3,
           Wr4, Wn4, b4, Wr5, Wn5, b5, Wl, bl):
    src = g[0].astype(jnp.int32)
    dst = g[1].astype(jnp.int32)
    pad = EP - E
    srcp = jnp.concatenate([src, jnp.zeros((pad,), jnp.int32)]).reshape(NW * CPW, CHUNK)
    dstp = jnp.concatenate([dst, jnp.full((pad,), N, jnp.int32)]).reshape(NW * CPW, CHUNK)
    z = jnp.zeros((NPAD, D), jnp.float32)
    xp = jnp.concatenate([x, jnp.zeros((NPAD - N, D), jnp.float32)])

    wlp = jnp.zeros((D, D), jnp.float32).at[:, :C].set(Wl)
    blp = jnp.full((D,), -1e30, jnp.float32).at[:C].set(bl)

    x1, s1 = _conv(_sc_agg(xp, srcp, dstp, z), xp, Wn1, Wr1, b1, pv=p1)
    _, X2, km1 = _pool(x1, s1, K1)

    x2, s2 = _conv(_sc_agg(X2, srcp, dstp, z), X2, Wn2, Wr2, b2, m=km1, pv=p2)
    s2m = jnp.where(km1[:, :1] > 0.0, s2, -jnp.inf)
    _, X3, km2 = _pool(x2, s2m, K2, s_prev=s1[:, 0])

    u1 = _conv(_sc_agg(X3, srcp, dstp, z), X3, Wn3, Wr3, b3, m=km2, r=x2)
    u2 = _conv(_sc_agg(u1, srcp, dstp, z), u1, Wn4, Wr4, b4, m=km1, r=x1)
    out = _conv(_sc_agg(u2, srcp, dstp, z), u2, Wn5, Wr5, b5, wl=wlp, blv=blp)
    return out[:N, :C]


# speedup vs baseline: 1.0634x; 1.0634x over previous
import jax
import jax.numpy as jnp
from jax import lax
from jax.experimental import pallas as pl
from jax.experimental.pallas import tpu as pltpu
from jax.experimental.pallas import tpu_sc as plsc

N, D, C = 10000, 128, 10
E = 320000
NW = 32
CHUNK = 128
CPW = 80
EP = NW * CPW * CHUNK
NPAD = 10240
BR = 1000
GRID = N // BR
K1 = 8000
K2 = 6400


def _sc_agg_body(x_hbm, src_hbm, dst_hbm, z_hbm, out_hbm,
                 acc, src_v, dst_v, rows0, rows1, sem0, sem1):
    cid = lax.axis_index("c")
    sid = lax.axis_index("s")
    wid = sid * 2 + cid
    HC = CPW // 2
    pltpu.sync_copy(src_hbm.at[pl.ds(wid * CPW, HC)], src_v)
    pltpu.sync_copy(dst_hbm.at[pl.ds(wid * CPW, HC)], dst_v)
    pltpu.async_copy(x_hbm.at[src_v.at[0]], rows0, sem0)
    zr = NPAD // 16
    pltpu.sync_copy(z_hbm.at[pl.ds(sid * zr, zr)], acc.at[pl.ds(sid * zr, zr)])
    plsc.subcore_barrier()
    for h in range(2):
        if h == 1:
            pltpu.sync_copy(src_hbm.at[pl.ds(wid * CPW + HC, HC)], src_v)
            pltpu.sync_copy(dst_hbm.at[pl.ds(wid * CPW + HC, HC)], dst_v)
            pltpu.async_copy(x_hbm.at[src_v.at[0]], rows0, sem0)

        def pair(i, carry):
            j0 = 2 * i
            j1 = j0 + 1
            pltpu.async_copy(x_hbm.at[src_v.at[j1]], rows1, sem1)
            pltpu.make_async_copy(x_hbm.at[src_v.at[j0]], rows0, sem0).wait()
            pltpu.sync_copy(rows0, acc.at[dst_v.at[j0]], add=True)
            jn = jnp.where(j0 + 2 < HC, j0 + 2, 0)
            pltpu.async_copy(x_hbm.at[src_v.at[jn]], rows0, sem0)
            pltpu.make_async_copy(x_hbm.at[src_v.at[j1]], rows1, sem1).wait()
            pltpu.sync_copy(rows1, acc.at[dst_v.at[j1]], add=True)
            return carry

        lax.fori_loop(0, HC // 2, pair, 0)
        pltpu.make_async_copy(x_hbm.at[src_v.at[0]], rows0, sem0).wait()
    plsc.subcore_barrier()
    pltpu.sync_copy(acc.at[pl.ds(sid * zr, zr)],
                    out_hbm.at[pl.ds(cid * NPAD + sid * zr, zr)])


_sc_agg_fn = None


def _sc_agg(x, srcp, dstp, z):
    global _sc_agg_fn
    if _sc_agg_fn is None:
        _sc_agg_fn = pl.kernel(
            _sc_agg_body,
            out_type=jax.ShapeDtypeStruct((2 * NPAD, D), jnp.float32),
            mesh=plsc.VectorSubcoreMesh(core_axis_name="c", subcore_axis_name="s"),
            scratch_types=[
                pltpu.VMEM_SHARED((NPAD, D), jnp.float32),
                pltpu.VMEM((CPW // 2, CHUNK), jnp.int32),
                pltpu.VMEM((CPW // 2, CHUNK), jnp.int32),
                pltpu.VMEM((CHUNK, D), jnp.float32),
                pltpu.VMEM((CHUNK, D), jnp.float32),
                pltpu.SemaphoreType.DMA,
                pltpu.SemaphoreType.DMA,
            ],
        )
    return _sc_agg_fn(x, srcp, dstp, z)


def _u32_key(s):
    i = lax.bitcast_convert_type(s, jnp.int32)
    u = lax.bitcast_convert_type(s, jnp.uint32)
    return jnp.where(i < 0, ~u, u | jnp.uint32(0x80000000))


def _make_conv_body(mask, resid, score, final):
    def body(*refs):
        it = iter(refs)
        a0 = next(it)[...]
        a1 = next(it)[...]
        x = next(it)[...]
        wn = next(it)[...]
        wr = next(it)[...]
        b = next(it)[...]
        m = next(it)[...] if mask else None
        r = next(it)[...] if resid else None
        pm = next(it)[...] if score else None
        pn = next(it)[...] if score else None
        wl = next(it)[...] if final else None
        blv = next(it)[...] if final else None
        out_ref = next(it)
        s_ref = next(it) if score else None
        z = jnp.dot(a0 + a1, wn, preferred_element_type=jnp.float32)
        z = z + jnp.dot(x, wr, preferred_element_type=jnp.float32)
        z = jnp.maximum(z + b[0:1, :], 0.0)
        if mask:
            z = z * m
        if resid:
            z = z + r
        if final:
            lg = jnp.dot(z, wl, preferred_element_type=jnp.float32) + blv[0:1, :]
            lg = lg - jnp.max(lg, axis=-1, keepdims=True)
            z = lg - jnp.log(jnp.sum(jnp.exp(lg), axis=-1, keepdims=True))
        out_ref[...] = z
        if score:
            s_ref[...] = jnp.tanh(
                jnp.dot(z, pm, preferred_element_type=jnp.float32) / pn[0:1, :])
    return body


_ROW = pl.BlockSpec((BR, D), lambda i: (i, 0))
_W = pl.BlockSpec((D, D), lambda i: (0, 0))
_B8 = pl.BlockSpec((8, D), lambda i: (0, 0))


def _conv(P, x, wn, wr, b, m=None, r=None, pv=None, wl=None, blv=None):
    score, final = pv is not None, wl is not None
    ins = [P[:N], P[NPAD:NPAD + N], x, wn, wr, jnp.broadcast_to(b[None, :], (8, D))]
    specs = [_ROW, _ROW, _ROW, _W, _W, _B8]
    if m is not None:
        ins.append(m); specs.append(_ROW)
    if r is not None:
        ins.append(r); specs.append(_ROW)
    if score:
        ins.append(jnp.broadcast_to(pv[:, None], (D, D)))
        ins.append(jnp.broadcast_to(jnp.linalg.norm(pv)[None, None], (8, D)))
        specs.extend([_W, _B8])
    if final:
        ins.extend([wl, jnp.broadcast_to(blv[None, :], (8, D))])
        specs.extend([_W, _B8])
    out_shape = jax.ShapeDtypeStruct((N, D), jnp.float32)
    out_spec = _ROW
    if score:
        out_shape = [out_shape, jax.ShapeDtypeStruct((N, D), jnp.float32)]
        out_spec = [_ROW, _ROW]
    return pl.pallas_call(
        _make_conv_body(m is not None, r is not None, score, final),
        grid=(GRID,),
        in_specs=specs,
        out_specs=out_spec,
        out_shape=out_shape,
    )(*ins)


def _select_body(s_ref, k_ref, t_ref):
    keys = _u32_key(s_ref[...])
    kk = k_ref[0, 0]

    def step(t, prefix):
        cand = prefix | (jnp.uint32(1) << (jnp.uint32(31) - t.astype(jnp.uint32)))
        cnt = jnp.sum((keys >= cand).astype(jnp.int32))
        return jnp.where(cnt >= kk, cand, prefix)

    tk = lax.fori_loop(0, 32, step, jnp.uint32(0))
    t_ref[0, 0] = lax.bitcast_convert_type(tk, jnp.int32)


def _select(s, kk):
    sp = jnp.concatenate([s, jnp.full((80 * 128 - N,), -jnp.inf, jnp.float32)])
    kk_arr = jnp.asarray(kk, jnp.int32).reshape(1, 1)
    return pl.pallas_call(
        _select_body,
        in_specs=[pl.BlockSpec(memory_space=pltpu.VMEM),
                  pl.BlockSpec(memory_space=pltpu.SMEM)],
        out_specs=pl.BlockSpec(memory_space=pltpu.SMEM),
        out_shape=jax.ShapeDtypeStruct((1, 1), jnp.int32),
    )(sp.reshape(80, 128), kk_arr)[0, 0]


def _scale_body(x_ref, rs_ref, o_ref):
    o_ref[...] = x_ref[...] * rs_ref[...]


def _scale(x, rs):
    return pl.pallas_call(
        _scale_body,
        grid=(GRID,),
        in_specs=[_ROW, _ROW],
        out_specs=_ROW,
        out_shape=jax.ShapeDtypeStruct((N, D), jnp.float32),
    )(x, rs)


def _pool(xout, sout, kk, s_prev=None):
    s = sout[:, 0]
    keys = _u32_key(s)
    tu = lax.bitcast_convert_type(_select(s, kk), jnp.uint32)
    gt = keys > tu
    eq = keys == tu
    need = kk - jnp.sum(gt.astype(jnp.int32))
    if s_prev is None:
        kept = gt | (eq & (jnp.cumsum(eq.astype(jnp.int32)) <= need))
    else:
        sp = jnp.where(eq, s_prev, -jnp.inf)
        kp = _u32_key(sp)
        t1 = lax.bitcast_convert_type(_select(sp, need), jnp.uint32)
        gt2 = kp > t1
        eq2 = kp == t1
        need2 = need - jnp.sum(gt2.astype(jnp.int32))
        kept = gt | gt2 | (eq2 & (jnp.cumsum(eq2.astype(jnp.int32)) <= need2))
    rs = jnp.where(kept[:, None], sout, 0.0)
    xn = _scale(xout, rs)
    km = jnp.broadcast_to(kept[:, None].astype(jnp.float32), (N, D))
    return kept, xn, km


def kernel(x, g, Wr1, Wn1, b1, p1, Wr2, Wn2, b2, p2, Wr3, Wn3, b3,
           Wr4, Wn4, b4, Wr5, Wn5, b5, Wl, bl):
    src = g[0].astype(jnp.int32)
    dst = g[1].astype(jnp.int32)
    pad = EP - E
    srcp = jnp.concatenate([src, jnp.zeros((pad,), jnp.int32)]).reshape(NW * CPW, CHUNK)
    ddst = N + lax.rem(lax.iota(jnp.int32, pad), jnp.int32(NPAD - N))
    dstp = jnp.concatenate([dst, ddst]).reshape(NW * CPW, CHUNK)
    z = jnp.zeros((NPAD, D), jnp.float32)

    wlp = jnp.zeros((D, D), jnp.float32).at[:, :C].set(Wl)
    blp = jnp.full((D,), -1e30, jnp.float32).at[:C].set(bl)

    x1, s1 = _conv(_sc_agg(x, srcp, dstp, z), x, Wn1, Wr1, b1, pv=p1)
    _, X2, km1 = _pool(x1, s1, K1)

    x2, s2 = _conv(_sc_agg(X2, srcp, dstp, z), X2, Wn2, Wr2, b2, m=km1, pv=p2)
    s2m = jnp.where(km1[:, :1] > 0.0, s2, -jnp.inf)
    _, X3, km2 = _pool(x2, s2m, K2, s_prev=s1[:, 0])

    u1 = _conv(_sc_agg(X3, srcp, dstp, z), X3, Wn3, Wr3, b3, m=km2, r=x2)
    u2 = _conv(_sc_agg(u1, srcp, dstp, z), u1, Wn4, Wr4, b4, m=km1, r=x1)
    out = _conv(_sc_agg(u2, srcp, dstp, z), u2, Wn5, Wr5, b5, wl=wlp, blv=blp)
    return out[:, :C]
